# baseline (speedup 1.0000x reference)
import jax
import jax.numpy as jnp
from jax import lax
from jax.experimental import pallas as pl
from jax.experimental.pallas import tpu as pltpu
from jax.experimental.pallas import tpu_sc as plsc

_D, _P = 128, 16384
_MARGIN = 0.5
_EPS2 = 1e-16

_NC, _NS, _L = 2, 16, 16
_NW = _NC * _NS
_BPW = _P // _NW
_CHUNK = 128
_NCHUNK = _BPW // _CHUNK
_GROUPS = _CHUNK // _L


def _rsqrt(x):
    y = plsc.bitcast(x, jnp.int32)
    y = jnp.int32(0x5F3759DF) - (y >> 1)
    r = plsc.bitcast(y, jnp.float32)
    for _ in range(3):
        r = r * (1.5 - 0.5 * x * r * r)
    return r


def _body(emb, i1, i2, tgt, out, idx1_v, idx2_v, tgt_v,
          e1a_v, e2a_v, e1b_v, e2b_v, dotp_v, n1p_v, n2p_v, acc_v,
          sem_a, sem_b):
    wid = lax.axis_index("c") * _NS + lax.axis_index("s")
    base = wid * _BPW
    bufs = [(e1a_v, e2a_v, sem_a), (e1b_v, e2b_v, sem_b)]

    cpi1 = pltpu.async_copy(i1.at[pl.ds(base, _BPW)], idx1_v, sem_a)
    cpi2 = pltpu.async_copy(i2.at[pl.ds(base, _BPW)], idx2_v, sem_a)
    cpt = pltpu.async_copy(tgt.at[pl.ds(base, _BPW)], tgt_v, sem_b)
    cpi1.wait()
    cpi2.wait()

    def fire(ch):
        e1_v, e2_v, sem = bufs[ch % 2]
        cp1 = pltpu.async_copy(
            emb.at[idx1_v.at[pl.ds(ch * _CHUNK, _CHUNK)]], e1_v, sem)
        cp2 = pltpu.async_copy(
            emb.at[idx2_v.at[pl.ds(ch * _CHUNK, _CHUNK)]], e2_v, sem)
        return cp1, cp2

    pending = fire(0)
    cpt.wait()
    acc = jnp.zeros((_L,), jnp.float32)
    for ch in range(_NCHUNK):
        e1_v, e2_v, _ = bufs[ch % 2]
        pending[0].wait()
        pending[1].wait()
        if ch + 1 < _NCHUNK:
            pending = fire(ch + 1)

        def sgroup_body(sg, acc, e1_v=e1_v, e2_v=e2_v, ch=ch):
            @plsc.parallel_loop(0, 4 * _L, step=1, unroll=4)
            def pair_body(j):
                p = sg * (4 * _L) + j
                dot = jnp.zeros((_L,), jnp.float32)
                n1 = jnp.zeros((_L,), jnp.float32)
                n2 = jnp.zeros((_L,), jnp.float32)
                for k in range(_D // _L):
                    v1 = e1_v[p, pl.ds(k * _L, _L)]
                    v2 = e2_v[p, pl.ds(k * _L, _L)]
                    dot = dot + v1 * v2
                    n1 = n1 + v1 * v1
                    n2 = n2 + v2 * v2
                dotp_v[j, pl.ds(0, _L)] = dot
                n1p_v[j, pl.ds(0, _L)] = n1
                n2p_v[j, pl.ds(0, _L)] = n2

            for h in range(4):
                rows = lax.iota(jnp.int32, _L) + h * _L
                zero = jnp.zeros((_L,), jnp.float32)

                @plsc.parallel_loop(0, _L, step=1, unroll=4,
                                    carry=(zero, zero, zero))
                def col_body(c, carry, rows=rows):
                    dotv, n1v, n2v = carry
                    cols = jnp.full((_L,), 0, jnp.int32) + c
                    dotv = dotv + plsc.load_gather(dotp_v, [rows, cols])
                    n1v = n1v + plsc.load_gather(n1p_v, [rows, cols])
                    n2v = n2v + plsc.load_gather(n2p_v, [rows, cols])
                    return (dotv, n1v, n2v)

                dotv, n1v, n2v = col_body
                r = _rsqrt(jnp.maximum(n1v, _EPS2) * jnp.maximum(n2v, _EPS2))
                dist = 1.0 - dotv * r
                t = tgt_v[pl.ds(ch * _CHUNK + sg * 4 * _L + h * _L, _L)]
                v = t * dist + (1.0 - t) * jnp.maximum(0.0, _MARGIN - dist)
                acc = acc + (0.5 / _P) * (v * v)
            return acc

        acc = lax.fori_loop(0, _GROUPS // 4, sgroup_body, acc)
    acc_v[...] = acc
    pltpu.sync_copy(acc_v, out.at[wid])


def _sc_loss(embeddings, i1, i2, tgt):
    mesh = plsc.VectorSubcoreMesh(
        core_axis_name="c", subcore_axis_name="s",
        num_cores=_NC, num_subcores=_NS)
    f = pl.kernel(
        _body,
        out_type=jax.ShapeDtypeStruct((_NW, _L), jnp.float32),
        mesh=mesh,
        compiler_params=pltpu.CompilerParams(needs_layout_passes=False),
        scratch_types=[
            pltpu.VMEM((_BPW,), jnp.int32),
            pltpu.VMEM((_BPW,), jnp.int32),
            pltpu.VMEM((_BPW,), jnp.float32),
            pltpu.VMEM((_CHUNK, _D), jnp.float32),
            pltpu.VMEM((_CHUNK, _D), jnp.float32),
            pltpu.VMEM((_CHUNK, _D), jnp.float32),
            pltpu.VMEM((_CHUNK, _D), jnp.float32),
            pltpu.VMEM((4 * _L, _L + 1), jnp.float32),
            pltpu.VMEM((4 * _L, _L + 1), jnp.float32),
            pltpu.VMEM((4 * _L, _L + 1), jnp.float32),
            pltpu.VMEM((_L,), jnp.float32),
            pltpu.SemaphoreType.DMA,
            pltpu.SemaphoreType.DMA,
        ],
    )
    return f(embeddings, i1, i2, tgt)


def kernel(embeddings, indices):
    i1 = indices[:, 0]
    i2 = indices[:, 1]
    tgt = indices[:, 2].astype(jnp.float32)
    partials = _sc_loss(embeddings, i1, i2, tgt)
    return jnp.sum(partials)


# speedup vs baseline: 1.0171x; 1.0171x over previous
import jax
import jax.numpy as jnp
from jax import lax
from jax.experimental import pallas as pl
from jax.experimental.pallas import tpu as pltpu
from jax.experimental.pallas import tpu_sc as plsc

_D, _P = 128, 16384
_MARGIN = 0.5
_EPS2 = 1e-16

_NC, _NS, _L = 2, 16, 16
_NW = _NC * _NS
_BPW = _P // _NW
_CHUNK = 128
_NCHUNK = _BPW // _CHUNK
_GROUPS = _CHUNK // _L


def _rsqrt(x):
    y = plsc.bitcast(x, jnp.int32)
    y = jnp.int32(0x5F3759DF) - (y >> 1)
    r = plsc.bitcast(y, jnp.float32)
    for _ in range(3):
        r = r * (1.5 - 0.5 * x * r * r)
    return r


def _body(emb, i1, i2, tgt, out, idx1_v, idx2_v, tgt_v,
          e1a_v, e2a_v, e1b_v, e2b_v, dotp_v, n1p_v, n2p_v, acc_v,
          sem_a, sem_b):
    wid = lax.axis_index("c") * _NS + lax.axis_index("s")
    base = wid * _BPW
    bufs = [(e1a_v, e2a_v, sem_a), (e1b_v, e2b_v, sem_b)]

    cpi1 = pltpu.async_copy(i1.at[pl.ds(base, _BPW)], idx1_v, sem_a)
    cpi2 = pltpu.async_copy(i2.at[pl.ds(base, _BPW)], idx2_v, sem_a)
    cpt = pltpu.async_copy(tgt.at[pl.ds(base, _BPW)], tgt_v, sem_b)
    cpi1.wait()
    cpi2.wait()

    def fire(ch):
        e1_v, e2_v, sem = bufs[ch % 2]
        cp1 = pltpu.async_copy(
            emb.at[idx1_v.at[pl.ds(ch * _CHUNK, _CHUNK)]], e1_v, sem)
        cp2 = pltpu.async_copy(
            emb.at[idx2_v.at[pl.ds(ch * _CHUNK, _CHUNK)]], e2_v, sem)
        return cp1, cp2

    pending = fire(0)
    cpt.wait()
    acc = jnp.zeros((_L,), jnp.float32)
    for ch in range(_NCHUNK):
        e1_v, e2_v, _ = bufs[ch % 2]
        pending[0].wait()
        pending[1].wait()
        if ch + 1 < _NCHUNK:
            pending = fire(ch + 1)

        def sgroup_body(sg, acc, e1_v=e1_v, e2_v=e2_v, ch=ch):
            @plsc.parallel_loop(0, 2 * _L, step=1, unroll=4)
            def pair_body(j):
                p = sg * (2 * _L) + j
                dot = jnp.zeros((_L,), jnp.float32)
                n1 = jnp.zeros((_L,), jnp.float32)
                n2 = jnp.zeros((_L,), jnp.float32)
                for k in range(_D // _L):
                    v1 = e1_v[p, pl.ds(k * _L, _L)]
                    v2 = e2_v[p, pl.ds(k * _L, _L)]
                    dot = dot + v1 * v2
                    n1 = n1 + v1 * v1
                    n2 = n2 + v2 * v2
                dotp_v[j, pl.ds(0, _L)] = dot
                n1p_v[j, pl.ds(0, _L)] = n1
                n2p_v[j, pl.ds(0, _L)] = n2

            for h in range(2):
                rows = lax.iota(jnp.int32, _L) + h * _L
                zero = jnp.zeros((_L,), jnp.float32)

                @plsc.parallel_loop(0, _L, step=1, unroll=4,
                                    carry=(zero, zero, zero))
                def col_body(c, carry, rows=rows):
                    dotv, n1v, n2v = carry
                    cols = jnp.full((_L,), 0, jnp.int32) + c
                    dotv = dotv + plsc.load_gather(dotp_v, [rows, cols])
                    n1v = n1v + plsc.load_gather(n1p_v, [rows, cols])
                    n2v = n2v + plsc.load_gather(n2p_v, [rows, cols])
                    return (dotv, n1v, n2v)

                dotv, n1v, n2v = col_body
                r = _rsqrt(jnp.maximum(n1v, _EPS2) * jnp.maximum(n2v, _EPS2))
                dist = 1.0 - dotv * r
                t = tgt_v[pl.ds(ch * _CHUNK + sg * 2 * _L + h * _L, _L)]
                v = t * dist + (1.0 - t) * jnp.maximum(0.0, _MARGIN - dist)
                acc = acc + (0.5 / _P) * (v * v)
            return acc

        acc = lax.fori_loop(0, _GROUPS // 2, sgroup_body, acc)
    acc_v[...] = acc
    pltpu.sync_copy(acc_v, out.at[wid])


def _sc_loss(embeddings, i1, i2, tgt):
    mesh = plsc.VectorSubcoreMesh(
        core_axis_name="c", subcore_axis_name="s",
        num_cores=_NC, num_subcores=_NS)
    f = pl.kernel(
        _body,
        out_type=jax.ShapeDtypeStruct((_NW, _L), jnp.float32),
        mesh=mesh,
        compiler_params=pltpu.CompilerParams(needs_layout_passes=False),
        scratch_types=[
            pltpu.VMEM((_BPW,), jnp.int32),
            pltpu.VMEM((_BPW,), jnp.int32),
            pltpu.VMEM((_BPW,), jnp.float32),
            pltpu.VMEM((_CHUNK, _D), jnp.float32),
            pltpu.VMEM((_CHUNK, _D), jnp.float32),
            pltpu.VMEM((_CHUNK, _D), jnp.float32),
            pltpu.VMEM((_CHUNK, _D), jnp.float32),
            pltpu.VMEM((2 * _L, _L + 1), jnp.float32),
            pltpu.VMEM((2 * _L, _L + 1), jnp.float32),
            pltpu.VMEM((2 * _L, _L + 1), jnp.float32),
            pltpu.VMEM((_L,), jnp.float32),
            pltpu.SemaphoreType.DMA,
            pltpu.SemaphoreType.DMA,
        ],
    )
    return f(embeddings, i1, i2, tgt)


def kernel(embeddings, indices):
    i1 = indices[:, 0]
    i2 = indices[:, 1]
    tgt = indices[:, 2].astype(jnp.float32)
    partials = _sc_loss(embeddings, i1, i2, tgt)
    return jnp.sum(partials)
